# baseline (speedup 1.0000x reference)
import functools

import jax
import jax.numpy as jnp
from jax import lax
from jax.experimental import pallas as pl
from jax.experimental.pallas import tpu as pltpu
from jax.experimental.pallas import tpu_sc as plsc

N = 100000
E = 200000
H = 64
NW = 32
EPT = 6272
E_PAD = NW * EPT
BN = 2048
GRID = (N + BN - 1) // BN
CN = 4000
NCH = N // CN
CB = 896
NQB = EPT // CB
NSUB = CB // 128


def _tc_body(coords_ref, feid_ref, isa_ref, mask_ref, fext_ref, bcd_ref,
             bcr_ref, w1_ref, b1_ref, wh_ref, ws_ref, whs_ref, wt_ref,
             bias_ref, ff_ref, nd_ref, code_ref, acc_ref):
    i = pl.program_id(0)
    nid = i * BN + lax.broadcasted_iota(jnp.int32, (BN, 1), 0)
    vrow = nid < N

    z = jnp.dot(coords_ref[...], w1_ref[...],
                preferred_element_type=jnp.float32) + b1_ref[...]
    h = jnp.tanh(z)
    s = 1.0 - h * h
    hs = h * s
    t = s * (1.0 - 3.0 * h * h)
    out = (jnp.dot(h, wh_ref[...], preferred_element_type=jnp.float32)
           + jnp.dot(s, ws_ref[...], preferred_element_type=jnp.float32)
           + jnp.dot(hs, whs_ref[...], preferred_element_type=jnp.float32)
           + jnp.dot(t, wt_ref[...], preferred_element_type=jnp.float32)
           + bias_ref[...])
    ff16 = out[:, 0:16]
    ff_ref[...] = ff16
    nd_ref[...] = out[:, 16:24]

    feid = feid_ref[...]
    isa = isa_ref[...]
    fmask = mask_ref[...]
    code_ref[...] = jnp.where(fmask > 0.5, feid | (isa << 18),
                              jnp.int32(-1))

    fext = fext_ref[...]
    rx = ff16[:, 0:1] + ff16[:, 4:5] + ff16[:, 8:9] + ff16[:, 12:13] - fext[:, 0:1]
    ry = ff16[:, 1:2] + ff16[:, 5:6] + ff16[:, 9:10] + ff16[:, 13:14] - fext[:, 1:2]
    rz = ff16[:, 2:3] + ff16[:, 6:7] + ff16[:, 10:11] + ff16[:, 14:15] - fext[:, 2:3]
    res_sq = rx * rx + ry * ry + rz * rz
    bcd = bcd_ref[...]
    bcr = bcr_ref[...]
    free_n = vrow & (bcd < 0.5)
    eq_num = jnp.sum(jnp.where(free_n, res_sq, 0.0))
    eq_cnt = jnp.sum(jnp.where(free_n, 1.0, 0.0))

    fr_num = jnp.float32(0.0)
    fr_cnt = jnp.float32(0.0)
    for f in range(4):
        ffsq = (ff16[:, 4 * f:4 * f + 1] ** 2
                + ff16[:, 4 * f + 1:4 * f + 2] ** 2
                + ff16[:, 4 * f + 2:4 * f + 3] ** 2)
        free_f = vrow & (fmask[:, f:f + 1] < 0.5)
        fr_num += jnp.sum(jnp.where(free_f, ffsq, 0.0))
        fr_cnt += jnp.sum(jnp.where(free_f, 1.0, 0.0))

    dx = out[:, 24:25]
    dy = out[:, 25:26]
    dz = out[:, 26:27]
    sd = vrow & (bcd > 0.5)
    sr = vrow & (bcr > 0.5)
    sup_x = jnp.sum(jnp.where(sd, dx * dx, 0.0))
    sup_y = jnp.sum(jnp.where(sd, dy * dy, 0.0))
    sup_c = jnp.sum(jnp.where(sd, 1.0, 0.0))
    sup_z = jnp.sum(jnp.where(sr, dz * dz, 0.0))
    sup_rc = jnp.sum(jnp.where(sr, 1.0, 0.0))

    scalars = [eq_num, eq_cnt, fr_num, fr_cnt, sup_x, sup_y, sup_c,
               sup_z, sup_rc]
    rio = lax.broadcasted_iota(jnp.int32, (8, 128), 0)
    lio = lax.broadcasted_iota(jnp.int32, (8, 128), 1)
    contrib = jnp.zeros((8, 128), jnp.float32)
    for k, v in enumerate(scalars):
        contrib += jnp.where((rio == 0) & (lio == k), v, 0.0)

    @pl.when(i == 0)
    def _():
        acc_ref[...] = jnp.zeros_like(acc_ref)

    acc_ref[...] += contrib


def _tc_dense(coords8, feid, isa, fmask, fext, bcd, bcr,
              w1p, b1r, wh, ws, whs, wt, bias32):
    full = lambda shp: pl.BlockSpec(shp, lambda i: (0, 0))
    row = lambda w: pl.BlockSpec((BN, w), lambda i: (i, 0))
    return pl.pallas_call(
        _tc_body,
        grid=(GRID,),
        in_specs=[row(8), row(4), row(4), row(4), row(3), row(1), row(1),
                  full((8, H)), full((1, H)), full((H, 32)), full((H, 32)),
                  full((H, 32)), full((H, 32)), full((1, 32))],
        out_specs=[row(16), row(8), row(4),
                   pl.BlockSpec((8, 128), lambda i: (0, 0))],
        out_shape=[jax.ShapeDtypeStruct((N, 16), jnp.float32),
                   jax.ShapeDtypeStruct((N, 8), jnp.float32),
                   jax.ShapeDtypeStruct((N, 4), jnp.int32),
                   jax.ShapeDtypeStruct((8, 128), jnp.float32)],
    )(coords8, feid, isa, fmask, fext, bcd, bcr,
      w1p, b1r, wh, ws, whs, wt, bias32)


def _sc_body(code0, code1, code2, code3, ffv0, ffv1, ffv2, ffv3,
             nd8, edata, connA, connB, out_hbm,
             fAxT, fAyT, fAzT, fBxT, fByT, fBzT, cbuf, fbuf,
             cA_buf, cB_buf, ndA, ndB, ebuf, pbuf, sem2):
    wid = lax.axis_index("s") * 2 + lax.axis_index("c")
    lo = wid * EPT
    iota = lax.iota(jnp.int32, 16)
    zero16 = jnp.zeros((16,), jnp.float32)
    c0 = jnp.zeros((16,), jnp.int32)

    def init(i, _):
        fAxT[pl.ds(i * 16, 16)] = zero16
        fAyT[pl.ds(i * 16, 16)] = zero16
        fAzT[pl.ds(i * 16, 16)] = zero16
        fBxT[pl.ds(i * 16, 16)] = zero16
        fByT[pl.ds(i * 16, 16)] = zero16
        fBzT[pl.ds(i * 16, 16)] = zero16
        return 0
    lax.fori_loop(0, EPT // 16, init, 0)

    def scan_pass(code_hbm, ff_hbm):
        def chunk(c, _):
            pltpu.sync_copy(code_hbm.at[pl.ds(c * CN, CN)], cbuf)
            pltpu.sync_copy(ff_hbm.at[pl.ds(c * CN, CN)], fbuf)

            def vec(j, _):
                for u in range(2):
                    g = j * 2 + u
                    code = cbuf[pl.ds(g * 16, 16)]
                    valid = code >= 0
                    e = code & 0x3FFFF
                    inr = valid & (e >= lo) & (e < lo + EPT)
                    is_a = (code & (1 << 18)) != 0
                    mA = inr & is_a
                    mB = inr & (~is_a)
                    idx = e - lo
                    r16 = g * 16 + iota
                    fx = plsc.load_gather(fbuf, [r16, c0])
                    fy = plsc.load_gather(fbuf, [r16, c0 + 1])
                    fz = plsc.load_gather(fbuf, [r16, c0 + 2])
                    plsc.store_scatter(fAxT, [idx], fx, mask=mA)
                    plsc.store_scatter(fAyT, [idx], fy, mask=mA)
                    plsc.store_scatter(fAzT, [idx], fz, mask=mA)
                    plsc.store_scatter(fBxT, [idx], fx, mask=mB)
                    plsc.store_scatter(fByT, [idx], fy, mask=mB)
                    plsc.store_scatter(fBzT, [idx], fz, mask=mB)
                return 0
            lax.fori_loop(0, CN // 32, vec, 0)
            return 0
        lax.fori_loop(0, NCH, chunk, 0)

    scan_pass(code0, ffv0)
    scan_pass(code1, ffv1)
    scan_pass(code2, ffv2)
    scan_pass(code3, ffv3)

    def qchunk(q, accs):
        ebase = lo + q * CB

        pltpu.sync_copy(edata.at[pl.ds(ebase, CB)], ebuf)
        pltpu.sync_copy(connA.at[pl.ds(ebase, CB)], cA_buf)
        pltpu.sync_copy(connB.at[pl.ds(ebase, CB)], cB_buf)

        cops = []
        for j in range(NSUB):
            cops.append(pltpu.async_copy(
                nd8.at[cA_buf.at[pl.ds(j * 128, 128)]],
                ndA.at[pl.ds(j * 128, 128)], sem2))
            cops.append(pltpu.async_copy(
                nd8.at[cB_buf.at[pl.ds(j * 128, 128)]],
                ndB.at[pl.ds(j * 128, 128)], sem2))
        for cop in cops:
            cop.wait()

        def vec(sidx, acc):
            aN, aM, aV = acc
            r16 = sidx * 16 + iota
            toff = q * CB + sidx * 16
            emask = (ebase + sidx * 16 + iota) < E

            cosv = plsc.load_gather(ebuf, [r16, c0])
            sinv = plsc.load_gather(ebuf, [r16, c0 + 1])
            pE = plsc.load_gather(ebuf, [r16, c0 + 2])
            pA = plsc.load_gather(ebuf, [r16, c0 + 3])
            pI = plsc.load_gather(ebuf, [r16, c0 + 4])
            ea = pE * pA
            ei = pE * pI
            is_h = jnp.abs(cosv) > jnp.abs(sinv)
            sgn = jnp.where(is_h, 1.0, -1.0)

            fAx = fAxT[pl.ds(toff, 16)]
            fAy = fAyT[pl.ds(toff, 16)]
            fAz = fAzT[pl.ds(toff, 16)]
            fBx = fBxT[pl.ds(toff, 16)]
            fBy = fByT[pl.ds(toff, 16)]
            fBz = fBzT[pl.ds(toff, 16)]

            lAx = fAx * cosv + fAy * sinv
            lAy = -fAx * sinv + fAy * cosv
            lBx = fBx * cosv + fBy * sinv
            lBy = -fBx * sinv + fBy * cosv

            cs = jnp.where(is_h, c0, c0 + 1)
            cc = jnp.where(is_h, c0 + 2, c0 + 3)
            cd = jnp.where(is_h, c0 + 4, c0 + 5)
            stA = plsc.load_gather(ndA, [r16, cs])
            stB = plsc.load_gather(ndB, [r16, cs])
            cvA = plsc.load_gather(ndA, [r16, cc]) * sgn
            cvB = plsc.load_gather(ndB, [r16, cc]) * sgn
            d3A = plsc.load_gather(ndA, [r16, cd]) * sgn
            d3B = plsc.load_gather(ndB, [r16, cd]) * sgn

            def sq(u):
                return u * u
            tN = sq(lAx + ea * stA) + sq(lBx - ea * stB)
            tM = sq(fAz + ei * cvA) + sq(fBz - ei * cvB)
            tV = sq(lAy + ei * d3A) + sq(lBy - ei * d3B)
            return (aN + jnp.where(emask, tN, zero16),
                    aM + jnp.where(emask, tM, zero16),
                    aV + jnp.where(emask, tV, zero16))

        return lax.fori_loop(0, CB // 16, vec, accs)

    aN, aM, aV = lax.fori_loop(0, NQB, qchunk, (zero16, zero16, zero16))

    pbuf[pl.ds(0, 16)] = aN
    pbuf[pl.ds(16, 16)] = aM
    pbuf[pl.ds(32, 16)] = aV
    plsc.subcore_barrier()
    pltpu.sync_copy(pbuf, out_hbm.at[pl.ds(wid * 48, 48)])


def _sc_sparse(code0, code1, code2, code3, ffv0, ffv1, ffv2, ffv3,
               nd8, edata, connA, connB):
    mesh = plsc.VectorSubcoreMesh(core_axis_name="c", subcore_axis_name="s")
    fn = functools.partial(
        pl.kernel,
        out_type=jax.ShapeDtypeStruct((NW * 48,), jnp.float32),
        mesh=mesh,
        compiler_params=pltpu.CompilerParams(needs_layout_passes=False,
                                             use_tc_tiling_on_sc=False),
        scratch_types=[
            pltpu.VMEM((EPT,), jnp.float32),
            pltpu.VMEM((EPT,), jnp.float32),
            pltpu.VMEM((EPT,), jnp.float32),
            pltpu.VMEM((EPT,), jnp.float32),
            pltpu.VMEM((EPT,), jnp.float32),
            pltpu.VMEM((EPT,), jnp.float32),
            pltpu.VMEM((CN,), jnp.int32),
            pltpu.VMEM((CN, 4), jnp.float32),
            pltpu.VMEM((CB,), jnp.int32),
            pltpu.VMEM((CB,), jnp.int32),
            pltpu.VMEM((CB, 8), jnp.float32),
            pltpu.VMEM((CB, 8), jnp.float32),
            pltpu.VMEM((CB, 8), jnp.float32),
            pltpu.VMEM((48,), jnp.float32),
            pltpu.SemaphoreType.DMA,
        ],
    )(_sc_body)
    return fn(code0, code1, code2, code3, ffv0, ffv1, ffv2, ffv3,
              nd8, edata, connA, connB)


def kernel(coords, connectivity, face_element_id, face_is_A_end, face_mask,
           elem_directions, F_ext, bc_disp, bc_rot, prop_E, prop_A,
           prop_I22, W1, b1, W2, b2):
    f32 = jnp.float32

    w1p = jnp.zeros((8, H), f32).at[0:3, :].set(W1)
    b1r = b1.reshape(1, H)
    z1 = jnp.zeros((H, 1), f32)
    wh_ff = jnp.concatenate(
        [W2[:, 3:6], z1, W2[:, 6:9], z1, W2[:, 9:12], z1, W2[:, 12:15], z1],
        axis=1)
    wh = jnp.concatenate([wh_ff, jnp.zeros((H, 8), f32), W2[:, 0:3],
                          jnp.zeros((H, 5), f32)], axis=1)
    ws = jnp.zeros((H, 32), f32)
    ws = ws.at[:, 16].set(W2[:, 0] * W1[0, :]).at[:, 17].set(W2[:, 1] * W1[2, :])
    whs = jnp.zeros((H, 32), f32)
    whs = (whs.at[:, 18].set(-2.0 * W2[:, 1] * W1[0, :] ** 2)
              .at[:, 19].set(-2.0 * W2[:, 0] * W1[2, :] ** 2))
    wt = jnp.zeros((H, 32), f32)
    wt = (wt.at[:, 20].set(-2.0 * W2[:, 1] * W1[0, :] ** 3)
            .at[:, 21].set(-2.0 * W2[:, 0] * W1[2, :] ** 3))
    b2ff = jnp.concatenate(
        [b2[3:6], jnp.zeros((1,), f32), b2[6:9], jnp.zeros((1,), f32),
         b2[9:12], jnp.zeros((1,), f32), b2[12:15], jnp.zeros((1,), f32)])
    bias32 = jnp.concatenate([b2ff, jnp.zeros((8,), f32), b2[0:3],
                              jnp.zeros((5,), f32)]).reshape(1, 32)

    coords8 = jnp.zeros((N, 8), f32).at[:, 0:3].set(coords)

    ff16, nd8, code_nat, acc = _tc_dense(
        coords8, face_element_id.astype(jnp.int32),
        face_is_A_end.astype(jnp.int32), face_mask, F_ext, bc_disp, bc_rot,
        w1p, b1r, wh, ws, whs, wt, bias32)

    code4 = code_nat.T
    ffv = ff16.reshape(N, 4, 4).transpose(1, 0, 2)
    pad_e = lambda x: jnp.pad(x, (0, E_PAD - E))
    edata = jnp.stack(
        [elem_directions[:, 0], elem_directions[:, 2], prop_E, prop_A,
         prop_I22, jnp.zeros((E,), f32), jnp.zeros((E,), f32),
         jnp.zeros((E,), f32)], axis=1)
    edata = jnp.pad(edata, ((0, E_PAD - E), (0, 0)))
    connA = pad_e(connectivity[:, 0].astype(jnp.int32))
    connB = pad_e(connectivity[:, 1].astype(jnp.int32))

    partial = _sc_sparse(code4[0], code4[1], code4[2], code4[3],
                         ffv[0], ffv[1], ffv[2], ffv[3],
                         nd8, edata, connA, connB)

    a = acc[0]
    eq_num, eq_cnt, fr_num, fr_cnt = a[0], a[1], a[2], a[3]
    sup_x, sup_y, sup_c, sup_z, sup_rc = a[4], a[5], a[6], a[7], a[8]
    p = partial.reshape(NW, 3, 16).sum(axis=(0, 2))
    L_eq = eq_num / jnp.maximum(eq_cnt, 1.0)
    L_free = fr_num / jnp.maximum(3.0 * fr_cnt, 1.0)
    L_sup = (sup_x / jnp.maximum(sup_c, 1.0)
             + sup_y / jnp.maximum(sup_c, 1.0)
             + sup_z / jnp.maximum(sup_rc, 1.0))
    return L_eq + L_free + L_sup + (p[0] + p[1] + p[2]) / E


# speedup vs baseline: 1.0070x; 1.0070x over previous
import functools

import jax
import jax.numpy as jnp
from jax import lax
from jax.experimental import pallas as pl
from jax.experimental.pallas import tpu as pltpu
from jax.experimental.pallas import tpu_sc as plsc

N = 100000
E = 200000
H = 64
NW = 32
EPT = 6272
E_PAD = NW * EPT
BN = 2048
GRID = (N + BN - 1) // BN
CN = 5000
NCH = N // CN
CB = 896
NQB = EPT // CB
NSUB = CB // 128


def _tc_body(coords_ref, feid_ref, isa_ref, mask_ref, fext_ref, bcd_ref,
             bcr_ref, w1_ref, b1_ref, wh_ref, ws_ref, whs_ref, wt_ref,
             bias_ref, ff_ref, nd_ref, code_ref, acc_ref):
    i = pl.program_id(0)
    nid = i * BN + lax.broadcasted_iota(jnp.int32, (BN, 1), 0)
    vrow = nid < N

    z = jnp.dot(coords_ref[...], w1_ref[...],
                preferred_element_type=jnp.float32) + b1_ref[...]
    h = jnp.tanh(z)
    s = 1.0 - h * h
    hs = h * s
    t = s * (1.0 - 3.0 * h * h)
    out = (jnp.dot(h, wh_ref[...], preferred_element_type=jnp.float32)
           + jnp.dot(s, ws_ref[...], preferred_element_type=jnp.float32)
           + jnp.dot(hs, whs_ref[...], preferred_element_type=jnp.float32)
           + jnp.dot(t, wt_ref[...], preferred_element_type=jnp.float32)
           + bias_ref[...])
    ff16 = out[:, 0:16]
    ff_ref[...] = ff16
    nd_ref[...] = out[:, 16:24]

    feid = feid_ref[...]
    isa = isa_ref[...]
    fmask = mask_ref[...]
    code_ref[...] = jnp.where(fmask > 0.5, feid | (isa << 18),
                              jnp.int32(-1))

    fext = fext_ref[...]
    rx = ff16[:, 0:1] + ff16[:, 4:5] + ff16[:, 8:9] + ff16[:, 12:13] - fext[:, 0:1]
    ry = ff16[:, 1:2] + ff16[:, 5:6] + ff16[:, 9:10] + ff16[:, 13:14] - fext[:, 1:2]
    rz = ff16[:, 2:3] + ff16[:, 6:7] + ff16[:, 10:11] + ff16[:, 14:15] - fext[:, 2:3]
    res_sq = rx * rx + ry * ry + rz * rz
    bcd = bcd_ref[...]
    bcr = bcr_ref[...]
    free_n = vrow & (bcd < 0.5)
    eq_num = jnp.sum(jnp.where(free_n, res_sq, 0.0))
    eq_cnt = jnp.sum(jnp.where(free_n, 1.0, 0.0))

    fr_num = jnp.float32(0.0)
    fr_cnt = jnp.float32(0.0)
    for f in range(4):
        ffsq = (ff16[:, 4 * f:4 * f + 1] ** 2
                + ff16[:, 4 * f + 1:4 * f + 2] ** 2
                + ff16[:, 4 * f + 2:4 * f + 3] ** 2)
        free_f = vrow & (fmask[:, f:f + 1] < 0.5)
        fr_num += jnp.sum(jnp.where(free_f, ffsq, 0.0))
        fr_cnt += jnp.sum(jnp.where(free_f, 1.0, 0.0))

    dx = out[:, 24:25]
    dy = out[:, 25:26]
    dz = out[:, 26:27]
    sd = vrow & (bcd > 0.5)
    sr = vrow & (bcr > 0.5)
    sup_x = jnp.sum(jnp.where(sd, dx * dx, 0.0))
    sup_y = jnp.sum(jnp.where(sd, dy * dy, 0.0))
    sup_c = jnp.sum(jnp.where(sd, 1.0, 0.0))
    sup_z = jnp.sum(jnp.where(sr, dz * dz, 0.0))
    sup_rc = jnp.sum(jnp.where(sr, 1.0, 0.0))

    scalars = [eq_num, eq_cnt, fr_num, fr_cnt, sup_x, sup_y, sup_c,
               sup_z, sup_rc]
    rio = lax.broadcasted_iota(jnp.int32, (8, 128), 0)
    lio = lax.broadcasted_iota(jnp.int32, (8, 128), 1)
    contrib = jnp.zeros((8, 128), jnp.float32)
    for k, v in enumerate(scalars):
        contrib += jnp.where((rio == 0) & (lio == k), v, 0.0)

    @pl.when(i == 0)
    def _():
        acc_ref[...] = jnp.zeros_like(acc_ref)

    acc_ref[...] += contrib


def _tc_dense(coords8, feid, isa, fmask, fext, bcd, bcr,
              w1p, b1r, wh, ws, whs, wt, bias32):
    full = lambda shp: pl.BlockSpec(shp, lambda i: (0, 0))
    row = lambda w: pl.BlockSpec((BN, w), lambda i: (i, 0))
    return pl.pallas_call(
        _tc_body,
        grid=(GRID,),
        in_specs=[row(8), row(4), row(4), row(4), row(3), row(1), row(1),
                  full((8, H)), full((1, H)), full((H, 32)), full((H, 32)),
                  full((H, 32)), full((H, 32)), full((1, 32))],
        out_specs=[row(16), row(8), row(4),
                   pl.BlockSpec((8, 128), lambda i: (0, 0))],
        out_shape=[jax.ShapeDtypeStruct((N, 16), jnp.float32),
                   jax.ShapeDtypeStruct((N, 8), jnp.float32),
                   jax.ShapeDtypeStruct((N, 4), jnp.int32),
                   jax.ShapeDtypeStruct((8, 128), jnp.float32)],
    )(coords8, feid, isa, fmask, fext, bcd, bcr,
      w1p, b1r, wh, ws, whs, wt, bias32)


def _sc_body(code0, code1, code2, code3, ffv0, ffv1, ffv2, ffv3,
             nd8, edata, connA, connB, out_hbm,
             fAxT, fAyT, fAzT, fBxT, fByT, fBzT, cbuf, fbuf,
             cA_buf, cB_buf, ndA, ndB, ebuf, pbuf, sem2):
    wid = lax.axis_index("s") * 2 + lax.axis_index("c")
    lo = wid * EPT
    iota = lax.iota(jnp.int32, 16)
    zero16 = jnp.zeros((16,), jnp.float32)
    c0 = jnp.zeros((16,), jnp.int32)

    def init(i, _):
        fAxT[pl.ds(i * 16, 16)] = zero16
        fAyT[pl.ds(i * 16, 16)] = zero16
        fAzT[pl.ds(i * 16, 16)] = zero16
        fBxT[pl.ds(i * 16, 16)] = zero16
        fByT[pl.ds(i * 16, 16)] = zero16
        fBzT[pl.ds(i * 16, 16)] = zero16
        return 0
    lax.fori_loop(0, EPT // 16, init, 0)

    def scan_pass(code_hbm, ff_hbm):
        def chunk(c, _):
            pltpu.sync_copy(code_hbm.at[pl.ds(c * CN, CN)], cbuf)
            pltpu.sync_copy(ff_hbm.at[pl.ds(c * CN, CN)], fbuf)

            def vec(j, _):
                code = cbuf[pl.ds(j * 16, 16)]
                valid = code >= 0
                e = code & 0x3FFFF
                inr = valid & (e >= lo) & (e < lo + EPT)
                is_a = (code & (1 << 18)) != 0
                mA = inr & is_a
                mB = inr & (~is_a)
                idx = e - lo
                r16 = j * 16 + iota
                fx = plsc.load_gather(fbuf, [r16, c0])
                fy = plsc.load_gather(fbuf, [r16, c0 + 1])
                fz = plsc.load_gather(fbuf, [r16, c0 + 2])
                plsc.store_scatter(fAxT, [idx], fx, mask=mA)
                plsc.store_scatter(fAyT, [idx], fy, mask=mA)
                plsc.store_scatter(fAzT, [idx], fz, mask=mA)
                plsc.store_scatter(fBxT, [idx], fx, mask=mB)
                plsc.store_scatter(fByT, [idx], fy, mask=mB)
                plsc.store_scatter(fBzT, [idx], fz, mask=mB)
                return 0
            lax.fori_loop(0, CN // 16, vec, 0)
            return 0
        lax.fori_loop(0, NCH, chunk, 0)

    scan_pass(code0, ffv0)
    scan_pass(code1, ffv1)
    scan_pass(code2, ffv2)
    scan_pass(code3, ffv3)

    def qchunk(q, accs):
        ebase = lo + q * CB

        pltpu.sync_copy(edata.at[pl.ds(ebase, CB)], ebuf)
        pltpu.sync_copy(connA.at[pl.ds(ebase, CB)], cA_buf)
        pltpu.sync_copy(connB.at[pl.ds(ebase, CB)], cB_buf)

        cops = []
        for j in range(NSUB):
            cops.append(pltpu.async_copy(
                nd8.at[cA_buf.at[pl.ds(j * 128, 128)]],
                ndA.at[pl.ds(j * 128, 128)], sem2))
            cops.append(pltpu.async_copy(
                nd8.at[cB_buf.at[pl.ds(j * 128, 128)]],
                ndB.at[pl.ds(j * 128, 128)], sem2))
        for cop in cops:
            cop.wait()

        def vec(sidx, acc):
            aN, aM, aV = acc
            r16 = sidx * 16 + iota
            toff = q * CB + sidx * 16
            emask = (ebase + sidx * 16 + iota) < E

            cosv = plsc.load_gather(ebuf, [r16, c0])
            sinv = plsc.load_gather(ebuf, [r16, c0 + 1])
            pE = plsc.load_gather(ebuf, [r16, c0 + 2])
            pA = plsc.load_gather(ebuf, [r16, c0 + 3])
            pI = plsc.load_gather(ebuf, [r16, c0 + 4])
            ea = pE * pA
            ei = pE * pI
            is_h = jnp.abs(cosv) > jnp.abs(sinv)
            sgn = jnp.where(is_h, 1.0, -1.0)

            fAx = fAxT[pl.ds(toff, 16)]
            fAy = fAyT[pl.ds(toff, 16)]
            fAz = fAzT[pl.ds(toff, 16)]
            fBx = fBxT[pl.ds(toff, 16)]
            fBy = fByT[pl.ds(toff, 16)]
            fBz = fBzT[pl.ds(toff, 16)]

            lAx = fAx * cosv + fAy * sinv
            lAy = -fAx * sinv + fAy * cosv
            lBx = fBx * cosv + fBy * sinv
            lBy = -fBx * sinv + fBy * cosv

            cs = jnp.where(is_h, c0, c0 + 1)
            cc = jnp.where(is_h, c0 + 2, c0 + 3)
            cd = jnp.where(is_h, c0 + 4, c0 + 5)
            stA = plsc.load_gather(ndA, [r16, cs])
            stB = plsc.load_gather(ndB, [r16, cs])
            cvA = plsc.load_gather(ndA, [r16, cc]) * sgn
            cvB = plsc.load_gather(ndB, [r16, cc]) * sgn
            d3A = plsc.load_gather(ndA, [r16, cd]) * sgn
            d3B = plsc.load_gather(ndB, [r16, cd]) * sgn

            def sq(u):
                return u * u
            tN = sq(lAx + ea * stA) + sq(lBx - ea * stB)
            tM = sq(fAz + ei * cvA) + sq(fBz - ei * cvB)
            tV = sq(lAy + ei * d3A) + sq(lBy - ei * d3B)
            return (aN + jnp.where(emask, tN, zero16),
                    aM + jnp.where(emask, tM, zero16),
                    aV + jnp.where(emask, tV, zero16))

        return lax.fori_loop(0, CB // 16, vec, accs)

    aN, aM, aV = lax.fori_loop(0, NQB, qchunk, (zero16, zero16, zero16))

    pbuf[pl.ds(0, 16)] = aN
    pbuf[pl.ds(16, 16)] = aM
    pbuf[pl.ds(32, 16)] = aV
    plsc.subcore_barrier()
    pltpu.sync_copy(pbuf, out_hbm.at[pl.ds(wid * 48, 48)])


def _sc_sparse(code0, code1, code2, code3, ffv0, ffv1, ffv2, ffv3,
               nd8, edata, connA, connB):
    mesh = plsc.VectorSubcoreMesh(core_axis_name="c", subcore_axis_name="s")
    fn = functools.partial(
        pl.kernel,
        out_type=jax.ShapeDtypeStruct((NW * 48,), jnp.float32),
        mesh=mesh,
        compiler_params=pltpu.CompilerParams(needs_layout_passes=False,
                                             use_tc_tiling_on_sc=False),
        scratch_types=[
            pltpu.VMEM((EPT,), jnp.float32),
            pltpu.VMEM((EPT,), jnp.float32),
            pltpu.VMEM((EPT,), jnp.float32),
            pltpu.VMEM((EPT,), jnp.float32),
            pltpu.VMEM((EPT,), jnp.float32),
            pltpu.VMEM((EPT,), jnp.float32),
            pltpu.VMEM((CN,), jnp.int32),
            pltpu.VMEM((CN, 4), jnp.float32),
            pltpu.VMEM((CB,), jnp.int32),
            pltpu.VMEM((CB,), jnp.int32),
            pltpu.VMEM((CB, 8), jnp.float32),
            pltpu.VMEM((CB, 8), jnp.float32),
            pltpu.VMEM((CB, 8), jnp.float32),
            pltpu.VMEM((48,), jnp.float32),
            pltpu.SemaphoreType.DMA,
        ],
    )(_sc_body)
    return fn(code0, code1, code2, code3, ffv0, ffv1, ffv2, ffv3,
              nd8, edata, connA, connB)


def kernel(coords, connectivity, face_element_id, face_is_A_end, face_mask,
           elem_directions, F_ext, bc_disp, bc_rot, prop_E, prop_A,
           prop_I22, W1, b1, W2, b2):
    f32 = jnp.float32

    w1p = jnp.zeros((8, H), f32).at[0:3, :].set(W1)
    b1r = b1.reshape(1, H)
    z1 = jnp.zeros((H, 1), f32)
    wh_ff = jnp.concatenate(
        [W2[:, 3:6], z1, W2[:, 6:9], z1, W2[:, 9:12], z1, W2[:, 12:15], z1],
        axis=1)
    wh = jnp.concatenate([wh_ff, jnp.zeros((H, 8), f32), W2[:, 0:3],
                          jnp.zeros((H, 5), f32)], axis=1)
    ws = jnp.zeros((H, 32), f32)
    ws = ws.at[:, 16].set(W2[:, 0] * W1[0, :]).at[:, 17].set(W2[:, 1] * W1[2, :])
    whs = jnp.zeros((H, 32), f32)
    whs = (whs.at[:, 18].set(-2.0 * W2[:, 1] * W1[0, :] ** 2)
              .at[:, 19].set(-2.0 * W2[:, 0] * W1[2, :] ** 2))
    wt = jnp.zeros((H, 32), f32)
    wt = (wt.at[:, 20].set(-2.0 * W2[:, 1] * W1[0, :] ** 3)
            .at[:, 21].set(-2.0 * W2[:, 0] * W1[2, :] ** 3))
    b2ff = jnp.concatenate(
        [b2[3:6], jnp.zeros((1,), f32), b2[6:9], jnp.zeros((1,), f32),
         b2[9:12], jnp.zeros((1,), f32), b2[12:15], jnp.zeros((1,), f32)])
    bias32 = jnp.concatenate([b2ff, jnp.zeros((8,), f32), b2[0:3],
                              jnp.zeros((5,), f32)]).reshape(1, 32)

    coords8 = jnp.zeros((N, 8), f32).at[:, 0:3].set(coords)

    ff16, nd8, code_nat, acc = _tc_dense(
        coords8, face_element_id.astype(jnp.int32),
        face_is_A_end.astype(jnp.int32), face_mask, F_ext, bc_disp, bc_rot,
        w1p, b1r, wh, ws, whs, wt, bias32)

    code4 = code_nat.T
    ffv = ff16.reshape(N, 4, 4).transpose(1, 0, 2)
    pad_e = lambda x: jnp.pad(x, (0, E_PAD - E))
    edata = jnp.stack(
        [elem_directions[:, 0], elem_directions[:, 2], prop_E, prop_A,
         prop_I22, jnp.zeros((E,), f32), jnp.zeros((E,), f32),
         jnp.zeros((E,), f32)], axis=1)
    edata = jnp.pad(edata, ((0, E_PAD - E), (0, 0)))
    connA = pad_e(connectivity[:, 0].astype(jnp.int32))
    connB = pad_e(connectivity[:, 1].astype(jnp.int32))

    partial = _sc_sparse(code4[0], code4[1], code4[2], code4[3],
                         ffv[0], ffv[1], ffv[2], ffv[3],
                         nd8, edata, connA, connB)

    a = acc[0]
    eq_num, eq_cnt, fr_num, fr_cnt = a[0], a[1], a[2], a[3]
    sup_x, sup_y, sup_c, sup_z, sup_rc = a[4], a[5], a[6], a[7], a[8]
    p = partial.reshape(NW, 3, 16).sum(axis=(0, 2))
    L_eq = eq_num / jnp.maximum(eq_cnt, 1.0)
    L_free = fr_num / jnp.maximum(3.0 * fr_cnt, 1.0)
    L_sup = (sup_x / jnp.maximum(sup_c, 1.0)
             + sup_y / jnp.maximum(sup_c, 1.0)
             + sup_z / jnp.maximum(sup_rc, 1.0))
    return L_eq + L_free + L_sup + (p[0] + p[1] + p[2]) / E
